# Initial kernel scaffold; baseline (speedup 1.0000x reference)
import jax
import jax.numpy as jnp
from jax.experimental import pallas as pl


def kernel(x, k_list):
    raise NotImplementedError("write your pallas kernel here")


# speedup vs baseline: 6.2857x; 6.2857x over previous
import functools

import jax
import jax.numpy as jnp
from jax import lax
from jax.experimental import pallas as pl
from jax.experimental.pallas import tpu as pltpu
from jax.experimental.pallas import tpu_sc as plsc

SF = 2
B, C, T = 64, 256, 4096
R = 8
NBLK = T // 16
NC, NS = 2, 16
SAMPLES_PER_W = B // (NC * NS)


def _body(x_hbm, k_hbm, out_hbm, k_v, in_v, out_v):
    wid = lax.axis_index("s") * NC + lax.axis_index("c")
    pltpu.sync_copy(k_hbm, k_v)
    iota = lax.iota(jnp.int32, 16)

    for s in range(SAMPLES_PER_W):
        b = wid * SAMPLES_PER_W + s
        k_vec = plsc.load_gather(k_v, [jnp.full((16,), b, jnp.int32)]) - SF
        col0 = iota - k_vec

        def chunk_body(ci, carry, b=b, col0=col0):
            c0 = ci * R
            pltpu.sync_copy(x_hbm.at[b, pl.ds(c0, R), :], in_v)
            for r in range(R):
                rsplat = jnp.full((16,), r, jnp.int32)
                colL = jnp.where(col0 < 0, -col0, col0)
                out_v[r, pl.ds(0, 16)] = plsc.load_gather(in_v, [rsplat, colL])

                def blk(j, c, rsplat=rsplat, r=r, col0=col0):
                    col = col0 + j * 16
                    out_v[r, pl.ds(j * 16, 16)] = plsc.load_gather(
                        in_v, [rsplat, col])
                    return c

                lax.fori_loop(1, NBLK - 1, blk, 0, unroll=8)

                colR = col0 + (NBLK - 1) * 16
                colR = jnp.where(colR > T - 1, 2 * (T - 1) - colR, colR)
                out_v[r, pl.ds((NBLK - 1) * 16, 16)] = plsc.load_gather(
                    in_v, [rsplat, colR])
            pltpu.sync_copy(out_v, out_hbm.at[b, pl.ds(c0, R), :])
            return carry

        lax.fori_loop(0, C // R, chunk_body, 0)


@jax.jit
def kernel(x, k_list):
    mesh = plsc.VectorSubcoreMesh(core_axis_name="c", subcore_axis_name="s")
    run = pl.kernel(
        _body,
        out_type=jax.ShapeDtypeStruct((B, C, T), jnp.float32),
        mesh=mesh,
        scratch_types=[
            pltpu.VMEM((B,), jnp.int32),
            pltpu.VMEM((R, T), jnp.float32),
            pltpu.VMEM((R, T), jnp.float32),
        ],
        compiler_params=pltpu.CompilerParams(needs_layout_passes=False),
    )
    return run(x, k_list.astype(jnp.int32))
